# baseline (speedup 1.0000x reference)
import functools

import jax
import jax.numpy as jnp
from jax import lax
from jax.experimental import pallas as pl
from jax.experimental.pallas import tpu as pltpu
from jax.experimental.pallas import tpu_sc as plsc

jax.config.update("jax_default_matmul_precision", "float32")

_EN = 8192
_ED = 64
_BM = 256

_NC = 2
_NS = 16
_NW = _NC * _NS
_CHUNK = 128


def _bf(v):
    return v.astype(jnp.bfloat16).astype(jnp.float32)


def _tree_row_sum(s):
    w = s.shape[1]
    while w > 1:
        w //= 2
        s = s[:, :w] + s[:, w:]
    return s


_THETA = 1e-3


def _vq_tc_body(z_ref, et_ref, idx_ref, dsum_ref, isum_ref):
    z = z_ref[...]
    et = et_ref[...]
    z2 = _tree_row_sum(z * z)
    e2 = jnp.sum(et * et, axis=0, keepdims=True)
    dn = (((1,), (0,)), ((), ()))
    prec = lax.Precision.DEFAULT

    def dot(a, b):
        return lax.dot_general(a, b, dn, precision=prec,
                               preferred_element_type=jnp.float32)

    def argmin_cols(dis):
        m = jnp.min(dis, axis=1, keepdims=True)
        col = lax.broadcasted_iota(jnp.int32, dis.shape, 1)
        idx = jnp.min(jnp.where(dis == m, col, jnp.int32(2**31 - 1)),
                      axis=1, keepdims=True)
        return m, idx

    def write(m, idx):
        idx_ref[...] = idx
        dsum_ref[...] = jnp.sum(m, keepdims=True).reshape(1, 1, 1)
        isum_ref[...] = jnp.sum(idx.astype(jnp.float32),
                                keepdims=True).reshape(1, 1, 1)

    zh = _bf(z); zm = _bf(z - zh); zr = _bf(z - zh - zm)
    eh = _bf(et); em = _bf(et - eh); er = _bf(et - eh - em)
    mm3 = dot(zh, eh) + (dot(zh, em) + dot(zm, eh))
    dis3 = (z2 + e2) - 2.0 * mm3
    m3, idx3 = argmin_cols(dis3)
    write(m3, idx3)
    near = jnp.sum(jnp.where(dis3 < m3 + _THETA, 1, 0), axis=1, keepdims=True)
    ambiguous = jnp.max(near) >= 2

    @pl.when(ambiguous)
    def _refine():
        mm6 = mm3 + ((dot(zh, er) + dot(zr, eh)) + dot(zm, em))
        dis6 = (z2 + e2) - 2.0 * mm6
        m6, idx6 = argmin_cols(dis6)
        write(m6, idx6)


def _argmin_distances(z, et):
    m_total = z.shape[0]
    grid = (m_total // _BM,)
    return pl.pallas_call(
        _vq_tc_body,
        grid=grid,
        in_specs=[
            pl.BlockSpec((_BM, _ED), lambda i: (i, 0)),
            pl.BlockSpec((_ED, _EN), lambda i: (0, 0)),
        ],
        out_specs=[
            pl.BlockSpec((_BM, 1), lambda i: (i, 0)),
            pl.BlockSpec((1, 1, 1), lambda i: (i, 0, 0)),
            pl.BlockSpec((1, 1, 1), lambda i: (i, 0, 0)),
        ],
        out_shape=[
            jax.ShapeDtypeStruct((m_total, 1), jnp.int32),
            jax.ShapeDtypeStruct((m_total // _BM, 1, 1), jnp.float32),
            jax.ShapeDtypeStruct((m_total // _BM, 1, 1), jnp.float32),
        ],
    )(z, et)


def _make_sc_gather(b_total):
    b_per_w = b_total // _NW
    n_chunks = b_per_w // _CHUNK
    mesh = plsc.VectorSubcoreMesh(core_axis_name="c", subcore_axis_name="s")

    @functools.partial(
        pl.kernel,
        mesh=mesh,
        compiler_params=pltpu.CompilerParams(use_tc_tiling_on_sc=False),
        out_type=jax.ShapeDtypeStruct((b_total, _ED), jnp.float32),
        scratch_types=[
            pltpu.VMEM((n_chunks, _CHUNK), jnp.int32),
            pltpu.VMEM((b_per_w, _ED), jnp.float32),
            pltpu.SemaphoreType.DMA,
        ],
    )
    def gather_k(table_hbm, idx_hbm, out_hbm, idx_v, rows_v, sem):
        wid = lax.axis_index("s") * _NC + lax.axis_index("c")
        pltpu.sync_copy(idx_hbm.at[wid], idx_v)
        copies = []
        for j in range(n_chunks):
            copies.append(pltpu.async_copy(
                table_hbm.at[idx_v.at[j]],
                rows_v.at[pl.ds(j * _CHUNK, _CHUNK)],
                sem))
        for c in copies:
            c.wait()
        pltpu.sync_copy(rows_v, out_hbm.at[pl.ds(wid * b_per_w, b_per_w)])

    return gather_k


def kernel(x, embedding):
    m_total = x.shape[0] * x.shape[1]
    z = x.reshape(m_total, _ED)
    et = embedding.T
    idx2d, dsum, isum = _argmin_distances(z, et)
    idx = idx2d.reshape(m_total)

    idx3 = idx.reshape(_NW, m_total // _NW // _CHUNK, _CHUNK)
    zq = _make_sc_gather(m_total)(embedding, idx3)
    z_q = zq.reshape(x.shape)

    denom = jnp.float32(m_total * _ED)
    loss = (1.0 + 0.25) * jnp.sum(dsum) / denom
    e_min = jnp.sum(isum) / jnp.float32(m_total)
    perplexity = jnp.exp(-(e_min * jnp.log(e_min + 1e-10)))
    return (loss, z_q, perplexity, idx)


# speedup vs baseline: 1.0548x; 1.0548x over previous
import functools

import jax
import jax.numpy as jnp
from jax import lax
from jax.experimental import pallas as pl
from jax.experimental.pallas import tpu as pltpu
from jax.experimental.pallas import tpu_sc as plsc

jax.config.update("jax_default_matmul_precision", "float32")

_EN = 8192
_ED = 64
_BM = 1024

_NC = 2
_NS = 16
_NW = _NC * _NS
_CHUNK = 128


def _bf(v):
    return v.astype(jnp.bfloat16).astype(jnp.float32)


def _tree_row_sum(s):
    w = s.shape[1]
    while w > 1:
        w //= 2
        s = s[:, :w] + s[:, w:]
    return s


def _vq_tc_body(z_ref, et_ref, idx_ref, dsum_ref, isum_ref):
    z = z_ref[...]
    et = et_ref[...]
    z2 = _tree_row_sum(z * z)
    e2 = jnp.sum(et * et, axis=0, keepdims=True)
    dn = (((1,), (0,)), ((), ()))
    prec = lax.Precision.DEFAULT

    def dot(a, b):
        return lax.dot_general(a, b, dn, precision=prec,
                               preferred_element_type=jnp.float32)

    zh = _bf(z); zm = _bf(z - zh); zr = _bf(z - zh - zm)
    eh = _bf(et); em = _bf(et - eh); er = _bf(et - eh - em)
    mm = (dot(zh, eh) + (dot(zh, em) + dot(zm, eh))
          + ((dot(zh, er) + dot(zr, eh)) + dot(zm, em)))
    dis = (z2 + e2) - 2.0 * mm
    m = jnp.min(dis, axis=1, keepdims=True)
    col = lax.broadcasted_iota(jnp.int32, dis.shape, 1)
    idx = jnp.min(jnp.where(dis == m, col, jnp.int32(2**31 - 1)),
                  axis=1, keepdims=True)
    idx_ref[...] = idx
    dsum_ref[...] = jnp.sum(m, keepdims=True).reshape(1, 1, 1)
    isum_ref[...] = jnp.sum(idx.astype(jnp.float32), keepdims=True).reshape(1, 1, 1)


def _argmin_distances(z, et):
    m_total = z.shape[0]
    grid = (m_total // _BM,)
    return pl.pallas_call(
        _vq_tc_body,
        grid=grid,
        in_specs=[
            pl.BlockSpec((_BM, _ED), lambda i: (i, 0)),
            pl.BlockSpec((_ED, _EN), lambda i: (0, 0)),
        ],
        out_specs=[
            pl.BlockSpec((_BM, 1), lambda i: (i, 0)),
            pl.BlockSpec((1, 1, 1), lambda i: (i, 0, 0)),
            pl.BlockSpec((1, 1, 1), lambda i: (i, 0, 0)),
        ],
        out_shape=[
            jax.ShapeDtypeStruct((m_total, 1), jnp.int32),
            jax.ShapeDtypeStruct((m_total // _BM, 1, 1), jnp.float32),
            jax.ShapeDtypeStruct((m_total // _BM, 1, 1), jnp.float32),
        ],
    )(z, et)


def _make_sc_gather(b_total):
    b_per_w = b_total // _NW
    n_chunks = b_per_w // _CHUNK
    mesh = plsc.VectorSubcoreMesh(core_axis_name="c", subcore_axis_name="s")

    @functools.partial(
        pl.kernel,
        mesh=mesh,
        compiler_params=pltpu.CompilerParams(use_tc_tiling_on_sc=False),
        out_type=jax.ShapeDtypeStruct((b_total, _ED), jnp.float32),
        scratch_types=[
            pltpu.VMEM((n_chunks, _CHUNK), jnp.int32),
            pltpu.VMEM((b_per_w, _ED), jnp.float32),
            pltpu.SemaphoreType.DMA,
        ],
    )
    def gather_k(table_hbm, idx_hbm, out_hbm, idx_v, rows_v, sem):
        wid = lax.axis_index("s") * _NC + lax.axis_index("c")
        pltpu.sync_copy(idx_hbm.at[wid], idx_v)
        copies = []
        for j in range(n_chunks):
            copies.append(pltpu.async_copy(
                table_hbm.at[idx_v.at[j]],
                rows_v.at[pl.ds(j * _CHUNK, _CHUNK)],
                sem))
        for c in copies:
            c.wait()
        pltpu.sync_copy(rows_v, out_hbm.at[pl.ds(wid * b_per_w, b_per_w)])

    return gather_k


def kernel(x, embedding):
    m_total = x.shape[0] * x.shape[1]
    z = x.reshape(m_total, _ED)
    et = embedding.T
    idx2d, dsum, isum = _argmin_distances(z, et)
    idx = idx2d.reshape(m_total)

    idx3 = idx.reshape(_NW, m_total // _NW // _CHUNK, _CHUNK)
    zq = _make_sc_gather(m_total)(embedding, idx3)
    z_q = zq.reshape(x.shape)

    denom = jnp.float32(m_total * _ED)
    loss = (1.0 + 0.25) * jnp.sum(dsum) / denom
    e_min = jnp.sum(isum) / jnp.float32(m_total)
    perplexity = jnp.exp(-(e_min * jnp.log(e_min + 1e-10)))
    return (loss, z_q, perplexity, idx)


# speedup vs baseline: 1.0797x; 1.0236x over previous
import functools

import jax
import jax.numpy as jnp
from jax import lax
from jax.experimental import pallas as pl
from jax.experimental.pallas import tpu as pltpu
from jax.experimental.pallas import tpu_sc as plsc

jax.config.update("jax_default_matmul_precision", "float32")

_EN = 8192
_ED = 64
_BM = 512

_NC = 2
_NS = 16
_NW = _NC * _NS
_CHUNK = 128


def _bf(v):
    return v.astype(jnp.bfloat16).astype(jnp.float32)


def _tree_row_sum(s):
    w = s.shape[1]
    while w > 1:
        w //= 2
        s = s[:, :w] + s[:, w:]
    return s


def _vq_tc_body(z_ref, et_ref, idx_ref, dsum_ref, isum_ref):
    z = z_ref[...]
    et = et_ref[...]
    z2 = _tree_row_sum(z * z)
    e2 = jnp.sum(et * et, axis=0, keepdims=True)
    dn = (((1,), (0,)), ((), ()))
    prec = lax.Precision.DEFAULT

    def dot(a, b):
        return lax.dot_general(a, b, dn, precision=prec,
                               preferred_element_type=jnp.float32)

    zh = _bf(z); zm = _bf(z - zh); zr = _bf(z - zh - zm)
    eh = _bf(et); em = _bf(et - eh); er = _bf(et - eh - em)
    mm = (dot(zh, eh) + (dot(zh, em) + dot(zm, eh))
          + ((dot(zh, er) + dot(zr, eh)) + dot(zm, em)))
    dis = (z2 + e2) - 2.0 * mm
    m = jnp.min(dis, axis=1, keepdims=True)
    col = lax.broadcasted_iota(jnp.int32, dis.shape, 1)
    idx = jnp.min(jnp.where(dis == m, col, jnp.int32(2**31 - 1)),
                  axis=1, keepdims=True)
    idx_ref[...] = idx
    dsum_ref[...] = jnp.sum(m, keepdims=True).reshape(1, 1, 1)
    isum_ref[...] = jnp.sum(idx.astype(jnp.float32), keepdims=True).reshape(1, 1, 1)


def _argmin_distances(z, et):
    m_total = z.shape[0]
    grid = (m_total // _BM,)
    return pl.pallas_call(
        _vq_tc_body,
        grid=grid,
        in_specs=[
            pl.BlockSpec((_BM, _ED), lambda i: (i, 0)),
            pl.BlockSpec((_ED, _EN), lambda i: (0, 0)),
        ],
        out_specs=[
            pl.BlockSpec((_BM, 1), lambda i: (i, 0)),
            pl.BlockSpec((1, 1, 1), lambda i: (i, 0, 0)),
            pl.BlockSpec((1, 1, 1), lambda i: (i, 0, 0)),
        ],
        out_shape=[
            jax.ShapeDtypeStruct((m_total, 1), jnp.int32),
            jax.ShapeDtypeStruct((m_total // _BM, 1, 1), jnp.float32),
            jax.ShapeDtypeStruct((m_total // _BM, 1, 1), jnp.float32),
        ],
    )(z, et)


def _make_sc_gather(b_total):
    b_per_w = b_total // _NW
    n_chunks = b_per_w // _CHUNK
    mesh = plsc.VectorSubcoreMesh(core_axis_name="c", subcore_axis_name="s")

    @functools.partial(
        pl.kernel,
        mesh=mesh,
        compiler_params=pltpu.CompilerParams(use_tc_tiling_on_sc=False),
        out_type=jax.ShapeDtypeStruct((b_total, _ED), jnp.float32),
        scratch_types=[
            pltpu.VMEM((n_chunks, _CHUNK), jnp.int32),
            pltpu.VMEM((b_per_w, _ED), jnp.float32),
            pltpu.SemaphoreType.DMA,
        ],
    )
    def gather_k(table_hbm, idx_hbm, out_hbm, idx_v, rows_v, sem):
        wid = lax.axis_index("s") * _NC + lax.axis_index("c")
        pltpu.sync_copy(idx_hbm.at[wid], idx_v)
        copies = []
        for j in range(n_chunks):
            copies.append(pltpu.async_copy(
                table_hbm.at[idx_v.at[j]],
                rows_v.at[pl.ds(j * _CHUNK, _CHUNK)],
                sem))
        for c in copies:
            c.wait()
        pltpu.sync_copy(rows_v, out_hbm.at[pl.ds(wid * b_per_w, b_per_w)])

    return gather_k


def kernel(x, embedding):
    m_total = x.shape[0] * x.shape[1]
    z = x.reshape(m_total, _ED)
    et = embedding.T
    idx2d, dsum, isum = _argmin_distances(z, et)
    idx = idx2d.reshape(m_total)

    idx3 = idx.reshape(_NW, m_total // _NW // _CHUNK, _CHUNK)
    zq = _make_sc_gather(m_total)(embedding, idx3)
    z_q = zq.reshape(x.shape)

    denom = jnp.float32(m_total * _ED)
    loss = (1.0 + 0.25) * jnp.sum(dsum) / denom
    e_min = jnp.sum(isum) / jnp.float32(m_total)
    perplexity = jnp.exp(-(e_min * jnp.log(e_min + 1e-10)))
    return (loss, z_q, perplexity, idx)


# speedup vs baseline: 1.2975x; 1.2017x over previous
import functools

import jax
import jax.numpy as jnp
from jax import lax
from jax.experimental import pallas as pl
from jax.experimental.pallas import tpu as pltpu
from jax.experimental.pallas import tpu_sc as plsc

jax.config.update("jax_default_matmul_precision", "float32")

_EN = 8192
_ED = 64
_BM = 512

_NC = 2
_NS = 16
_NW = _NC * _NS
_CHUNK = 128


def _bf(v):
    return v.astype(jnp.bfloat16).astype(jnp.float32)


def _tree_row_sum(s):
    w = s.shape[1]
    while w > 1:
        w //= 2
        s = s[:, :w] + s[:, w:]
    return s


_THETA = 1e-3
_NREF = 1024


def _split3(v):
    h = _bf(v)
    m = _bf(v - h)
    r = _bf(v - h - m)
    return h, m, r


_DN = (((1,), (0,)), ((), ()))


def _dot(a, b):
    return lax.dot_general(a, b, _DN, precision=lax.Precision.DEFAULT,
                           preferred_element_type=jnp.float32)


def _argmin_cols(dis):
    m = jnp.min(dis, axis=1, keepdims=True)
    col = lax.broadcasted_iota(jnp.int32, dis.shape, 1)
    idx = jnp.min(jnp.where(dis == m, col, jnp.int32(2**31 - 1)),
                  axis=1, keepdims=True)
    return m, idx


def _vq_coarse_body(z_ref, et_ref, idx_ref, amb_ref, dsum_ref, isum_ref):
    z = z_ref[...]
    et = et_ref[...]
    z2 = _tree_row_sum(z * z)
    e2 = jnp.sum(et * et, axis=0, keepdims=True)
    zh, zm, _ = _split3(z)
    eh, em, _ = _split3(et)
    mm3 = _dot(zh, eh) + (_dot(zh, em) + _dot(zm, eh))
    dis3 = (z2 + e2) - 2.0 * mm3
    m3, idx3 = _argmin_cols(dis3)
    idx_ref[...] = idx3
    amb_ref[...] = jnp.sum(jnp.where(dis3 < m3 + _THETA, 1, 0),
                           axis=1, keepdims=True)
    dsum_ref[...] = jnp.sum(m3, keepdims=True).reshape(1, 1, 1)
    isum_ref[...] = jnp.sum(idx3.astype(jnp.float32),
                            keepdims=True).reshape(1, 1, 1)


def _vq_exact_body(z_ref, et_ref, idx_ref):
    z = z_ref[...]
    et = et_ref[...]
    z2 = _tree_row_sum(z * z)
    e2 = jnp.sum(et * et, axis=0, keepdims=True)
    zh, zm, zr = _split3(z)
    eh, em, er = _split3(et)
    mm = ((_dot(zh, eh) + (_dot(zh, em) + _dot(zm, eh)))
          + ((_dot(zh, er) + _dot(zr, eh)) + _dot(zm, em)))
    dis = (z2 + e2) - 2.0 * mm
    _, idx = _argmin_cols(dis)
    idx_ref[...] = idx


def _argmin_distances(z, et):
    m_total = z.shape[0]
    return pl.pallas_call(
        _vq_coarse_body,
        grid=(m_total // _BM,),
        in_specs=[
            pl.BlockSpec((_BM, _ED), lambda i: (i, 0)),
            pl.BlockSpec((_ED, _EN), lambda i: (0, 0)),
        ],
        out_specs=[
            pl.BlockSpec((_BM, 1), lambda i: (i, 0)),
            pl.BlockSpec((_BM, 1), lambda i: (i, 0)),
            pl.BlockSpec((1, 1, 1), lambda i: (i, 0, 0)),
            pl.BlockSpec((1, 1, 1), lambda i: (i, 0, 0)),
        ],
        out_shape=[
            jax.ShapeDtypeStruct((m_total, 1), jnp.int32),
            jax.ShapeDtypeStruct((m_total, 1), jnp.int32),
            jax.ShapeDtypeStruct((m_total // _BM, 1, 1), jnp.float32),
            jax.ShapeDtypeStruct((m_total // _BM, 1, 1), jnp.float32),
        ],
    )(z, et)


def _refine_ambiguous(zref_rows, et):
    return pl.pallas_call(
        _vq_exact_body,
        grid=(_NREF // _BM,),
        in_specs=[
            pl.BlockSpec((_BM, _ED), lambda i: (i, 0)),
            pl.BlockSpec((_ED, _EN), lambda i: (0, 0)),
        ],
        out_specs=pl.BlockSpec((_BM, 1), lambda i: (i, 0)),
        out_shape=jax.ShapeDtypeStruct((_NREF, 1), jnp.int32),
    )(zref_rows, et)


def _make_sc_gather(b_total):
    b_per_w = b_total // _NW
    n_chunks = b_per_w // _CHUNK
    mesh = plsc.VectorSubcoreMesh(core_axis_name="c", subcore_axis_name="s")

    @functools.partial(
        pl.kernel,
        mesh=mesh,
        compiler_params=pltpu.CompilerParams(use_tc_tiling_on_sc=False),
        out_type=jax.ShapeDtypeStruct((b_total, _ED), jnp.float32),
        scratch_types=[
            pltpu.VMEM((n_chunks, _CHUNK), jnp.int32),
            pltpu.VMEM((b_per_w, _ED), jnp.float32),
            pltpu.SemaphoreType.DMA,
        ],
    )
    def gather_k(table_hbm, idx_hbm, out_hbm, idx_v, rows_v, sem):
        wid = lax.axis_index("s") * _NC + lax.axis_index("c")
        pltpu.sync_copy(idx_hbm.at[wid], idx_v)
        copies = []
        for j in range(n_chunks):
            copies.append(pltpu.async_copy(
                table_hbm.at[idx_v.at[j]],
                rows_v.at[pl.ds(j * _CHUNK, _CHUNK)],
                sem))
        for c in copies:
            c.wait()
        pltpu.sync_copy(rows_v, out_hbm.at[pl.ds(wid * b_per_w, b_per_w)])

    return gather_k


def kernel(x, embedding):
    m_total = x.shape[0] * x.shape[1]
    z = x.reshape(m_total, _ED)
    et = embedding.T
    idx2d, amb2d, dsum, isum = _argmin_distances(z, et)
    idx3 = idx2d.reshape(m_total)
    amb_score = (amb2d.reshape(m_total) >= 2).astype(jnp.float32)
    ref_ids = lax.top_k(amb_score, _NREF)[1]
    idx_ref_rows = _refine_ambiguous(z[ref_ids], et).reshape(_NREF)
    idx = idx3.at[ref_ids].set(idx_ref_rows)

    idx3 = idx.reshape(_NW, m_total // _NW // _CHUNK, _CHUNK)
    zq = _make_sc_gather(m_total)(embedding, idx3)
    z_q = zq.reshape(x.shape)

    denom = jnp.float32(m_total * _ED)
    loss = (1.0 + 0.25) * jnp.sum(dsum) / denom
    e_min = jnp.sum(isum) / jnp.float32(m_total)
    perplexity = jnp.exp(-(e_min * jnp.log(e_min + 1e-10)))
    return (loss, z_q, perplexity, idx)
